# Initial kernel scaffold; baseline (speedup 1.0000x reference)
import jax
import jax.numpy as jnp
from jax.experimental import pallas as pl


def kernel(inputs, weight):
    raise NotImplementedError("write your pallas kernel here")


# speedup vs baseline: 1.0221x; 1.0221x over previous
import functools

import jax
import jax.numpy as jnp
from jax import lax
from jax.experimental import pallas as pl
from jax.experimental.pallas import tpu as pltpu
from jax.experimental.pallas import tpu_sc as plsc

NC, NS = 2, 16
NW = NC * NS
C = 128
D = 32


@functools.partial(jax.jit, static_argnames=("total_chunks",))
def _gather(table, idx2d, total_chunks):
    ch_per_w = total_chunks // NW
    mesh = plsc.VectorSubcoreMesh(core_axis_name="c", subcore_axis_name="s")

    @functools.partial(
        pl.kernel,
        out_type=jax.ShapeDtypeStruct((total_chunks * C, D), jnp.float32),
        mesh=mesh,
        scratch_types=[
            pltpu.VMEM((ch_per_w, C), jnp.int32),
            pltpu.VMEM((C, D), jnp.float32),
            pltpu.SemaphoreType.DMA,
        ],
        compiler_params=pltpu.CompilerParams(use_tc_tiling_on_sc=False),
    )
    def k(table_hbm, idx_hbm, out_hbm, idx_v, rows_v, sem):
        wid = lax.axis_index("s") * NC + lax.axis_index("c")
        base_chunk = wid * ch_per_w
        pltpu.sync_copy(idx_hbm.at[pl.ds(base_chunk, ch_per_w)], idx_v)

        def step(j, carry):
            pltpu.async_copy(table_hbm.at[idx_v.at[j]], rows_v, sem).wait()
            pltpu.sync_copy(
                rows_v, out_hbm.at[pl.ds((base_chunk + j) * C, C)]
            )
            return carry

        lax.fori_loop(0, ch_per_w, step, 0)

    return k(table, idx2d)


def kernel(inputs, weight):
    original_shape = inputs.shape
    flat = inputs.reshape(-1).astype(jnp.int32)
    total = flat.shape[0]
    total_chunks = total // (NW * C) * NW
    main = total_chunks * C
    idx2d = flat[:main].reshape(total_chunks, C)
    out = _gather(weight, idx2d, total_chunks)
    if main != total:
        tail = jnp.take(weight, flat[main:], axis=0)
        out = jnp.concatenate([out, tail], axis=0)
    return out.reshape(original_shape + (weight.shape[1],))


# speedup vs baseline: 1.1124x; 1.0883x over previous
import functools

import jax
import jax.numpy as jnp
from jax import lax
from jax.experimental import pallas as pl
from jax.experimental.pallas import tpu as pltpu
from jax.experimental.pallas import tpu_sc as plsc

NC, NS = 2, 16
NW = NC * NS
C = 128
GROUP = 10
D = 32
BANK_ROWS = GROUP * C


@functools.partial(jax.jit, static_argnames=("total_chunks",))
def _gather(table, idx2d, total_chunks):
    ch_per_w = total_chunks // NW
    n_groups = ch_per_w // GROUP
    assert n_groups * GROUP == ch_per_w and n_groups % 2 == 0
    total_rows = total_chunks * C
    mesh = plsc.VectorSubcoreMesh(core_axis_name="c", subcore_axis_name="s")

    @functools.partial(
        pl.kernel,
        out_type=jax.ShapeDtypeStruct((total_rows, D), jnp.float32),
        mesh=mesh,
        scratch_types=[
            pltpu.VMEM((ch_per_w, C), jnp.int32),
            pltpu.VMEM((BANK_ROWS, D), jnp.float32),
            pltpu.VMEM((BANK_ROWS, D), jnp.float32),
            pltpu.SemaphoreType.DMA,
            pltpu.SemaphoreType.DMA,
            pltpu.SemaphoreType.DMA,
            pltpu.SemaphoreType.DMA,
        ],
        compiler_params=pltpu.CompilerParams(use_tc_tiling_on_sc=False),
    )
    def k(table_hbm, idx_hbm, out_hbm, idx_v, rows0, rows1, gs0, gs1, os0, os1):
        wid = lax.axis_index("s") * NC + lax.axis_index("c")
        base_chunk = wid * ch_per_w
        pltpu.sync_copy(idx_hbm.at[pl.ds(base_chunk, ch_per_w)], idx_v)

        def fire_gathers(g, bank, sem):
            for b in range(GROUP):
                pltpu.async_copy(
                    table_hbm.at[idx_v.at[g * GROUP + b]],
                    bank.at[pl.ds(b * C, C)],
                    sem,
                )

        def drain_gathers(bank, sem):
            pltpu.make_async_copy(
                table_hbm.at[pl.ds(0, BANK_ROWS)], bank, sem
            ).wait()

        def fire_out(g, bank, sem):
            pltpu.async_copy(
                bank,
                out_hbm.at[pl.ds((base_chunk + g * GROUP) * C, BANK_ROWS)],
                sem,
            )

        def drain_out(bank, sem):
            pltpu.make_async_copy(
                bank, out_hbm.at[pl.ds(0, BANK_ROWS)], sem
            ).wait()

        fire_gathers(0, rows0, gs0)

        def step(p, carry):
            g0 = 2 * p
            g1 = 2 * p + 1

            @pl.when(p > 0)
            def _():
                drain_out(rows1, os1)

            fire_gathers(g1, rows1, gs1)
            drain_gathers(rows0, gs0)
            fire_out(g0, rows0, os0)
            drain_out(rows0, os0)

            @pl.when(p < n_groups // 2 - 1)
            def _():
                fire_gathers(g0 + 2, rows0, gs0)

            drain_gathers(rows1, gs1)
            fire_out(g1, rows1, os1)
            return carry

        lax.fori_loop(0, n_groups // 2, step, 0)
        drain_out(rows1, os1)

    return k(table, idx2d)


def kernel(inputs, weight):
    original_shape = inputs.shape
    flat = inputs.reshape(-1).astype(jnp.int32)
    total = flat.shape[0]
    total_chunks = (total // (NW * C * GROUP * 2)) * NW * GROUP * 2
    main = total_chunks * C
    idx2d = flat[:main].reshape(total_chunks, C)
    out = _gather(weight, idx2d, total_chunks)
    if main != total:
        tail = jnp.take(weight, flat[main:], axis=0)
        out = jnp.concatenate([out, tail], axis=0)
    return out.reshape(original_shape + (weight.shape[1],))
